# baseline (speedup 1.0000x reference)
import jax
import jax.numpy as jnp
import numpy as np
from jax.experimental import pallas as pl
from jax.experimental.pallas import tpu as pltpu

_T = 256
_RB = 8
_KC = 64


def _tile_kernel(ij_ref, xiT_ref, xjT_ref, out_ref, acc_ref):
    t = pl.program_id(0)
    d = xiT_ref.shape[0]
    is_compute = ij_ref[2, t] == 1

    @pl.when(is_compute)
    def _compute():
        for a0 in range(0, _T, _RB):
            accs = [None] * _RB
            for k0 in range(0, d, _KC):
                slab = xjT_ref[k0 : k0 + _KC, :]
                slab3 = slab.reshape(_KC // 8, 8, _T)
                for r in range(_RB):
                    col = xiT_ref[k0 : k0 + _KC, a0 + r : a0 + r + 1]
                    dif = jnp.abs(slab - col).reshape(_KC // 8, 8, _T)
                    part = jnp.max(dif, axis=0)
                    accs[r] = part if accs[r] is None else jnp.maximum(accs[r], part)
            for r in range(_RB):
                out_row = jnp.max(accs[r], axis=0, keepdims=True)
                acc_ref[a0 + r : a0 + r + 1, :] = out_row
        out_ref[:, :] = acc_ref[:, :]

    @pl.when(jnp.logical_not(is_compute))
    def _mirror():
        out_ref[:, :] = acc_ref[:, :].T


def _pairwise_inf(xT, steps, n, d, interpret=False):
    nsteps = steps.shape[1]
    grid_spec = pltpu.PrefetchScalarGridSpec(
        num_scalar_prefetch=1,
        grid=(nsteps,),
        in_specs=[
            pl.BlockSpec((d, _T), lambda t, ij: (0, ij[0, t])),
            pl.BlockSpec((d, _T), lambda t, ij: (0, ij[1, t])),
        ],
        out_specs=pl.BlockSpec((_T, _T), lambda t, ij: (ij[3, t], ij[4, t])),
        scratch_shapes=[pltpu.VMEM((_T, _T), xT.dtype)],
    )
    return pl.pallas_call(
        _tile_kernel,
        grid_spec=grid_spec,
        out_shape=jax.ShapeDtypeStruct((n, n), xT.dtype),
        interpret=interpret,
    )(steps, xT, xT)


def _make_steps(nb):
    cols = []
    for i in range(nb):
        cols.append((i, i, 1, i, i))
        for j in range(i + 1, nb):
            cols.append((i, j, 1, i, j))
            cols.append((i, j, 0, j, i))
    return np.array(cols, dtype=np.int32).T


def kernel(x):
    n, d = x.shape
    steps = _make_steps(n // _T)
    return _pairwise_inf(x.T, jnp.asarray(steps), n, d)


# speedup vs baseline: 1.1378x; 1.1378x over previous
import jax
import jax.numpy as jnp
import numpy as np
from jax.experimental import pallas as pl
from jax.experimental.pallas import tpu as pltpu

_T = 256


def _tile_kernel(ij_ref, xiT_ref, xjT_ref, out_ref, acc_ref):
    t = pl.program_id(0)
    is_compute = ij_ref[2, t] == 1

    @pl.when(is_compute)
    def _compute():
        d = xiT_ref.shape[0]
        kc = 128
        for a in range(_T):
            col = xiT_ref[:, a : a + 1]
            m = None
            for k0 in range(0, d, kc):
                part = jnp.max(
                    jnp.abs(xjT_ref[k0 : k0 + kc, :] - col[k0 : k0 + kc, :]),
                    axis=0,
                    keepdims=True,
                )
                m = part if m is None else jnp.maximum(m, part)
            acc_ref[a : a + 1, :] = m
        out_ref[:, :] = acc_ref[:, :]

    @pl.when(jnp.logical_not(is_compute))
    def _mirror():
        out_ref[:, :] = acc_ref[:, :].T


def _pairwise_inf(xT, steps, n, d, interpret=False):
    nsteps = steps.shape[1]
    grid_spec = pltpu.PrefetchScalarGridSpec(
        num_scalar_prefetch=1,
        grid=(nsteps,),
        in_specs=[
            pl.BlockSpec((d, _T), lambda t, ij: (0, ij[0, t])),
            pl.BlockSpec((d, _T), lambda t, ij: (0, ij[1, t])),
        ],
        out_specs=pl.BlockSpec((_T, _T), lambda t, ij: (ij[3, t], ij[4, t])),
        scratch_shapes=[pltpu.VMEM((_T, _T), xT.dtype)],
    )
    return pl.pallas_call(
        _tile_kernel,
        grid_spec=grid_spec,
        out_shape=jax.ShapeDtypeStruct((n, n), xT.dtype),
        interpret=interpret,
    )(steps, xT, xT)


def _make_steps(nb):
    cols = []
    for i in range(nb):
        cols.append((i, i, 1, i, i))
        for j in range(i + 1, nb):
            cols.append((i, j, 1, i, j))
            cols.append((i, j, 0, j, i))
    return np.array(cols, dtype=np.int32).T


def kernel(x):
    n, d = x.shape
    steps = _make_steps(n // _T)
    return _pairwise_inf(x.T, jnp.asarray(steps), n, d)
